# Initial kernel scaffold; baseline (speedup 1.0000x reference)
import jax
import jax.numpy as jnp
from jax.experimental import pallas as pl


def kernel(x, edge_index, W1, b1, W2, b2):
    raise NotImplementedError("write your pallas kernel here")


# speedup vs baseline: 4.9492x; 4.9492x over previous
import functools

import jax
import jax.numpy as jnp
from jax import lax
from jax.experimental import pallas as pl
from jax.experimental.pallas import tpu as pltpu
from jax.experimental.pallas import tpu_sc as plsc

NC = 2
NS = 16
NW = NC * NS


def _sc_aggregate(support, src, dst):
    n_nodes, d = support.shape
    n_edges = src.shape[0]
    epw = n_edges // NW
    ch = 80
    n_chunks = epw // ch
    assert epw % ch == 0 and n_nodes % 16 == 0
    row_chunks = n_nodes // 16
    per_sub = -(-row_chunks // NS)

    mesh = plsc.VectorSubcoreMesh(core_axis_name="c", subcore_axis_name="s")

    @functools.partial(
        pl.kernel,
        out_type=jax.ShapeDtypeStruct((NC, n_nodes, d), jnp.float32),
        mesh=mesh,
        compiler_params=pltpu.CompilerParams(use_tc_tiling_on_sc=False),
        scratch_types=[
            pltpu.VMEM((ch,), jnp.int32),
            pltpu.VMEM((ch,), jnp.int32),
            pltpu.VMEM((ch, d), jnp.float32),
            pltpu.VMEM((16, d), jnp.float32),
            pltpu.VMEM_SHARED((n_nodes, d), jnp.float32),
            pltpu.SemaphoreType.DMA,
        ],
    )
    def agg_kernel(support_hbm, src_hbm, dst_hbm, out_hbm,
                   src_v, dst_v, rows_v, zero_v, acc_sh, sem):
        cid = lax.axis_index("c")
        sid = lax.axis_index("s")
        wid = sid * NC + cid

        @pl.loop(0, 16)
        def _(r):
            for cb in range(d // 16):
                zero_v[r, pl.ds(cb * 16, 16)] = jnp.zeros((16,), jnp.float32)

        @pl.loop(0, per_sub)
        def _(j):
            chunk = sid + j * NS

            @pl.when(chunk < row_chunks)
            def _():
                pltpu.sync_copy(zero_v, acc_sh.at[pl.ds(chunk * 16, 16)])

        plsc.subcore_barrier()

        ebase = wid * epw

        @pl.loop(0, n_chunks)
        def _(j):
            base = ebase + j * ch
            pltpu.sync_copy(src_hbm.at[pl.ds(base, ch)], src_v)
            pltpu.sync_copy(dst_hbm.at[pl.ds(base, ch)], dst_v)
            pltpu.async_copy(support_hbm.at[src_v], rows_v, sem).wait()
            pltpu.sync_copy(rows_v, acc_sh.at[dst_v], add=True)

        plsc.subcore_barrier()

        @pl.loop(0, per_sub)
        def _(j):
            chunk = sid + j * NS

            @pl.when(chunk < row_chunks)
            def _():
                pltpu.sync_copy(acc_sh.at[pl.ds(chunk * 16, 16)],
                                out_hbm.at[cid, pl.ds(chunk * 16, 16)])

    return agg_kernel(support, src, dst)


_BM = 2000


def _tc_matmul(x, w):
    m, k = x.shape
    n = w.shape[1]

    def body(x_ref, w_ref, o_ref):
        o_ref[...] = jnp.dot(x_ref[...], w_ref[...],
                             preferred_element_type=jnp.float32)

    return pl.pallas_call(
        body,
        grid=(m // _BM,),
        in_specs=[pl.BlockSpec((_BM, k), lambda i: (i, 0)),
                  pl.BlockSpec((k, n), lambda i: (0, 0))],
        out_specs=pl.BlockSpec((_BM, n), lambda i: (i, 0)),
        out_shape=jax.ShapeDtypeStruct((m, n), jnp.float32),
    )(x, w)


def _tc_relu_matmul(p0, p1, b, w):
    m, k = p0.shape
    n = w.shape[1]

    def body(p0_ref, p1_ref, b_ref, w_ref, o_ref):
        h = jnp.maximum(p0_ref[...] + p1_ref[...] + b_ref[...], 0.0)
        o_ref[...] = jnp.dot(h, w_ref[...], preferred_element_type=jnp.float32)

    return pl.pallas_call(
        body,
        grid=(m // _BM,),
        in_specs=[pl.BlockSpec((_BM, k), lambda i: (i, 0)),
                  pl.BlockSpec((_BM, k), lambda i: (i, 0)),
                  pl.BlockSpec((1, k), lambda i: (0, 0)),
                  pl.BlockSpec((k, n), lambda i: (0, 0))],
        out_specs=pl.BlockSpec((_BM, n), lambda i: (i, 0)),
        out_shape=jax.ShapeDtypeStruct((m, n), jnp.float32),
    )(p0, p1, b.reshape(1, k), w)


def _tc_log_softmax(p0, p1, b):
    m, n = p0.shape

    def body(p0_ref, p1_ref, b_ref, o_ref):
        z = p0_ref[...] + p1_ref[...] + b_ref[...]
        zs = z - jnp.max(z, axis=1, keepdims=True)
        o_ref[...] = zs - jnp.log(jnp.sum(jnp.exp(zs), axis=1, keepdims=True))

    return pl.pallas_call(
        body,
        grid=(m // _BM,),
        in_specs=[pl.BlockSpec((_BM, n), lambda i: (i, 0)),
                  pl.BlockSpec((_BM, n), lambda i: (i, 0)),
                  pl.BlockSpec((1, n), lambda i: (0, 0))],
        out_specs=pl.BlockSpec((_BM, n), lambda i: (i, 0)),
        out_shape=jax.ShapeDtypeStruct((m, n), jnp.float32),
    )(p0, p1, b.reshape(1, n))


def kernel(x, edge_index, W1, b1, W2, b2):
    src = edge_index[0]
    dst = edge_index[1]
    s1 = _tc_matmul(x, W1)
    p1 = _sc_aggregate(s1, src, dst)
    s2 = _tc_relu_matmul(p1[0], p1[1], b1, W2)
    p2 = _sc_aggregate(s2, src, dst)
    return _tc_log_softmax(p2[0], p2[1], b2)


# speedup vs baseline: 7.4360x; 1.5025x over previous
import functools

import jax
import jax.numpy as jnp
from jax import lax
from jax.experimental import pallas as pl
from jax.experimental.pallas import tpu as pltpu
from jax.experimental.pallas import tpu_sc as plsc

NC = 2
NS = 16
NW = NC * NS
CH = 112
RB = 80


def _sc_aggregate(support, src3, dst3, zblk, n_nodes, nbuf):
    d = support.shape[1]
    cpw = src3.shape[1]
    assert cpw > nbuf
    n_acc = n_nodes + RB
    zero_chunks = n_acc // RB
    out_chunks = n_nodes // RB
    per_sub = -(-zero_chunks // NS)

    mesh = plsc.VectorSubcoreMesh(core_axis_name="c", subcore_axis_name="s")

    @functools.partial(
        pl.kernel,
        out_type=jax.ShapeDtypeStruct((NC, n_nodes, d), jnp.float32),
        mesh=mesh,
        compiler_params=pltpu.CompilerParams(use_tc_tiling_on_sc=False),
        scratch_types=[
            pltpu.VMEM((cpw, CH), jnp.int32),
            pltpu.VMEM((cpw, CH), jnp.int32),
            pltpu.VMEM((nbuf, CH, d), jnp.float32),
            pltpu.VMEM_SHARED((n_acc, d), jnp.float32),
            [pltpu.SemaphoreType.DMA] * nbuf,
            [pltpu.SemaphoreType.DMA] * nbuf,
        ],
    )
    def agg_kernel(support_hbm, src_hbm, dst_hbm, zblk_hbm, out_hbm,
                   srcv, dstv, rows, acc_sh, gsems, ssems):
        cid = lax.axis_index("c")
        sid = lax.axis_index("s")
        wid = sid * NC + cid

        pltpu.sync_copy(src_hbm.at[wid], srcv)
        pltpu.sync_copy(dst_hbm.at[wid], dstv)

        @pl.loop(0, per_sub)
        def _(j):
            chunk = sid + j * NS

            @pl.when(chunk < zero_chunks)
            def _():
                pltpu.sync_copy(zblk_hbm, acc_sh.at[pl.ds(chunk * RB, RB)])

        plsc.subcore_barrier()

        def start_gather(c, b):
            pltpu.async_copy(support_hbm.at[srcv.at[c]], rows.at[b], gsems[b])

        def wait_gather(c, b):
            pltpu.make_async_copy(
                support_hbm.at[srcv.at[c]], rows.at[b], gsems[b]).wait()

        def start_scatter(c, b):
            pltpu.async_copy(rows.at[b], acc_sh.at[dstv.at[c]], ssems[b],
                             add=True)

        def wait_scatter(c, b):
            pltpu.make_async_copy(
                rows.at[b], acc_sh.at[dstv.at[c]], ssems[b]).wait()

        for b in range(nbuf):
            start_gather(b, b)

        n_outer = -(-cpw // nbuf)

        @pl.loop(0, n_outer)
        def _(j):
            for b in range(nbuf):
                c = j * nbuf + b
                prev = (b - 1) % nbuf

                @pl.when(c < cpw)
                def _():
                    wait_gather(c, b)
                    start_scatter(c, b)
                    nxt = c + nbuf - 1

                    @pl.when((c >= 1) & (nxt < cpw))
                    def _():
                        wait_scatter(c - 1, prev)
                        start_gather(nxt, prev)

        for b in range(nbuf):
            c_last = ((cpw - 1 - b) // nbuf) * nbuf + b
            wait_scatter(c_last, b)

        plsc.subcore_barrier()

        @pl.loop(0, per_sub)
        def _(j):
            chunk = sid + j * NS

            @pl.when(chunk < out_chunks)
            def _():
                pltpu.sync_copy(acc_sh.at[pl.ds(chunk * RB, RB)],
                                out_hbm.at[cid, pl.ds(chunk * RB, RB)])

    return agg_kernel(support, src3, dst3, zblk)


_BM = 2000


def _tc_matmul(x, w):
    m, k = x.shape
    n = w.shape[1]

    def body(x_ref, w_ref, o_ref):
        o_ref[...] = jnp.dot(x_ref[...], w_ref[...],
                             preferred_element_type=jnp.float32)

    return pl.pallas_call(
        body,
        grid=(m // _BM,),
        in_specs=[pl.BlockSpec((_BM, k), lambda i: (i, 0)),
                  pl.BlockSpec((k, n), lambda i: (0, 0))],
        out_specs=pl.BlockSpec((_BM, n), lambda i: (i, 0)),
        out_shape=jax.ShapeDtypeStruct((m, n), jnp.float32),
    )(x, w)


def _tc_relu_matmul(p0, p1, b, w):
    m, k = p0.shape
    n = w.shape[1]

    def body(p0_ref, p1_ref, b_ref, w_ref, o_ref):
        h = jnp.maximum(p0_ref[...] + p1_ref[...] + b_ref[...], 0.0)
        o_ref[...] = jnp.dot(h, w_ref[...], preferred_element_type=jnp.float32)

    return pl.pallas_call(
        body,
        grid=(m // _BM,),
        in_specs=[pl.BlockSpec((_BM, k), lambda i: (i, 0)),
                  pl.BlockSpec((_BM, k), lambda i: (i, 0)),
                  pl.BlockSpec((1, k), lambda i: (0, 0)),
                  pl.BlockSpec((k, n), lambda i: (0, 0))],
        out_specs=pl.BlockSpec((_BM, n), lambda i: (i, 0)),
        out_shape=jax.ShapeDtypeStruct((m, n), jnp.float32),
    )(p0, p1, b.reshape(1, k), w)


def _tc_log_softmax(p0, p1, b):
    m, n = p0.shape

    def body(p0_ref, p1_ref, b_ref, o_ref):
        z = p0_ref[...] + p1_ref[...] + b_ref[...]
        zs = z - jnp.max(z, axis=1, keepdims=True)
        o_ref[...] = zs - jnp.log(jnp.sum(jnp.exp(zs), axis=1, keepdims=True))

    return pl.pallas_call(
        body,
        grid=(m // _BM,),
        in_specs=[pl.BlockSpec((_BM, n), lambda i: (i, 0)),
                  pl.BlockSpec((_BM, n), lambda i: (i, 0)),
                  pl.BlockSpec((1, n), lambda i: (0, 0))],
        out_specs=pl.BlockSpec((_BM, n), lambda i: (i, 0)),
        out_shape=jax.ShapeDtypeStruct((m, n), jnp.float32),
    )(p0, p1, b.reshape(1, n))


def kernel(x, edge_index, W1, b1, W2, b2):
    n_nodes = x.shape[0]
    src = edge_index[0]
    dst = edge_index[1]
    n_edges = src.shape[0]

    cpw = -(-n_edges // (NW * CH))
    pad = NW * CH * cpw - n_edges
    src3 = jnp.concatenate(
        [src, jnp.zeros((pad,), jnp.int32)]).reshape(NW, cpw, CH)
    dst3 = jnp.concatenate(
        [dst, n_nodes + (jnp.arange(pad, dtype=jnp.int32) % RB)]
    ).reshape(NW, cpw, CH)
    z1 = jnp.zeros((RB, W1.shape[1]), jnp.float32)
    z2 = jnp.zeros((RB, W2.shape[1]), jnp.float32)

    s1 = _tc_matmul(x, W1)
    p1 = _sc_aggregate(s1, src3, dst3, z1, n_nodes, 2)
    s2 = _tc_relu_matmul(p1[0], p1[1], b1, W2)
    p2 = _sc_aggregate(s2, src3, dst3, z2, n_nodes, 4)
    return _tc_log_softmax(p2[0], p2[1], b2)


# speedup vs baseline: 12.3395x; 1.6594x over previous
import functools

import jax
import jax.numpy as jnp
from jax import lax
from jax.experimental import pallas as pl
from jax.experimental.pallas import tpu as pltpu
from jax.experimental.pallas import tpu_sc as plsc

NC = 2
NS = 16
NW = NC * NS
CH = 80
RB = 80
NBUF = 3


def _sc_aggregate(support, src3, dst3, zblk):
    n_nodes, d = support.shape
    cpw = src3.shape[1]
    assert cpw > NBUF and n_nodes % RB == 0
    chunks_n = n_nodes // RB
    per_sub = -(-chunks_n // NS)

    mesh = plsc.VectorSubcoreMesh(core_axis_name="c", subcore_axis_name="s")

    @functools.partial(
        pl.kernel,
        out_type=jax.ShapeDtypeStruct((NC, n_nodes, d), jnp.float32),
        mesh=mesh,
        compiler_params=pltpu.CompilerParams(use_tc_tiling_on_sc=False),
        scratch_types=[
            pltpu.VMEM((cpw, CH), jnp.int32),
            pltpu.VMEM((cpw, CH), jnp.int32),
            pltpu.VMEM((NBUF, CH, d), jnp.float32),
            pltpu.VMEM_SHARED((n_nodes, d), jnp.float32),
            [pltpu.SemaphoreType.DMA] * NBUF,
            [pltpu.SemaphoreType.DMA] * NBUF,
        ],
    )
    def agg_kernel(support_hbm, src_hbm, dst_hbm, zblk_hbm, out_hbm,
                   srcv, dstv, rows, acc_sh, gsems, ssems):
        cid = lax.axis_index("c")
        sid = lax.axis_index("s")
        wid = sid * NC + cid

        pltpu.sync_copy(src_hbm.at[wid], srcv)
        pltpu.sync_copy(dst_hbm.at[wid], dstv)

        @pl.loop(0, per_sub)
        def _(j):
            chunk = sid + j * NS

            @pl.when(chunk < chunks_n)
            def _():
                pltpu.sync_copy(zblk_hbm, acc_sh.at[pl.ds(chunk * RB, RB)])

        plsc.subcore_barrier()

        def start_gather(c, b):
            pltpu.async_copy(support_hbm.at[srcv.at[c]], rows.at[b], gsems[b])

        def wait_gather(c, b):
            pltpu.make_async_copy(
                support_hbm.at[srcv.at[c]], rows.at[b], gsems[b]).wait()

        def start_scatter(c, b):
            pltpu.async_copy(rows.at[b], acc_sh.at[dstv.at[c]], ssems[b],
                             add=True)

        def wait_scatter(c, b):
            pltpu.make_async_copy(
                rows.at[b], acc_sh.at[dstv.at[c]], ssems[b]).wait()

        for b in range(NBUF):
            start_gather(b, b)

        n_outer = -(-cpw // NBUF)

        @pl.loop(0, n_outer)
        def _(j):
            for b in range(NBUF):
                c = j * NBUF + b
                prev = (b - 1) % NBUF

                @pl.when(c < cpw)
                def _():
                    wait_gather(c, b)
                    start_scatter(c, b)
                    nxt = c + NBUF - 1

                    @pl.when((c >= 1) & (nxt < cpw))
                    def _():
                        wait_scatter(c - 1, prev)
                        start_gather(nxt, prev)

        for b in range(NBUF):
            c_last = ((cpw - 1 - b) // NBUF) * NBUF + b
            wait_scatter(c_last, b)

        plsc.subcore_barrier()

        @pl.loop(0, per_sub)
        def _(j):
            chunk = sid + j * NS

            @pl.when(chunk < chunks_n)
            def _():
                pltpu.sync_copy(acc_sh.at[pl.ds(chunk * RB, RB)],
                                out_hbm.at[cid, pl.ds(chunk * RB, RB)])

    return agg_kernel(support, src3, dst3, zblk)


_BM = 2000


def _tc_matmul(x, w):
    m, k = x.shape
    n = w.shape[1]

    def body(x_ref, w_ref, o_ref):
        o_ref[...] = jnp.dot(x_ref[...], w_ref[...],
                             preferred_element_type=jnp.float32)

    return pl.pallas_call(
        body,
        grid=(m // _BM,),
        in_specs=[pl.BlockSpec((_BM, k), lambda i: (i, 0)),
                  pl.BlockSpec((k, n), lambda i: (0, 0))],
        out_specs=pl.BlockSpec((_BM, n), lambda i: (i, 0)),
        out_shape=jax.ShapeDtypeStruct((m, n), jnp.float32),
    )(x, w)


def _tc_relu_matmul(p, b, w):
    m, k = p.shape[1:]
    n = w.shape[1]

    def body(p0_ref, p1_ref, b_ref, w_ref, o_ref):
        h = jnp.maximum(p0_ref[0] + p1_ref[0] + b_ref[...], 0.0)
        o_ref[...] = jnp.dot(h, w_ref[...], preferred_element_type=jnp.float32)

    return pl.pallas_call(
        body,
        grid=(m // _BM,),
        in_specs=[pl.BlockSpec((1, _BM, k), lambda i: (0, i, 0)),
                  pl.BlockSpec((1, _BM, k), lambda i: (1, i, 0)),
                  pl.BlockSpec((1, k), lambda i: (0, 0)),
                  pl.BlockSpec((k, n), lambda i: (0, 0))],
        out_specs=pl.BlockSpec((_BM, n), lambda i: (i, 0)),
        out_shape=jax.ShapeDtypeStruct((m, n), jnp.float32),
    )(p, p, b.reshape(1, k), w)


def _tc_log_softmax(p, b):
    m, n = p.shape[1:]

    def body(p0_ref, p1_ref, b_ref, o_ref):
        z = p0_ref[0] + p1_ref[0] + b_ref[...]
        zs = z - jnp.max(z, axis=1, keepdims=True)
        o_ref[...] = zs - jnp.log(jnp.sum(jnp.exp(zs), axis=1, keepdims=True))

    return pl.pallas_call(
        body,
        grid=(m // _BM,),
        in_specs=[pl.BlockSpec((1, _BM, n), lambda i: (0, i, 0)),
                  pl.BlockSpec((1, _BM, n), lambda i: (1, i, 0)),
                  pl.BlockSpec((1, n), lambda i: (0, 0))],
        out_specs=pl.BlockSpec((_BM, n), lambda i: (i, 0)),
        out_shape=jax.ShapeDtypeStruct((m, n), jnp.float32),
    )(p, p, b.reshape(1, n))


def kernel(x, edge_index, W1, b1, W2, b2):
    n_edges = edge_index.shape[1]
    cpw = n_edges // (NW * CH)
    assert cpw * NW * CH == n_edges
    src3 = edge_index[0].reshape(NW, cpw, CH)
    dst3 = edge_index[1].reshape(NW, cpw, CH)
    z1 = jnp.zeros((RB, W1.shape[1]), jnp.float32)
    z2 = jnp.zeros((RB, W2.shape[1]), jnp.float32)

    s1 = _tc_matmul(x, W1)
    p1 = _sc_aggregate(s1, src3, dst3, z1)
    s2 = _tc_relu_matmul(p1, b1, W2)
    p2 = _sc_aggregate(s2, src3, dst3, z2)
    return _tc_log_softmax(p2, b2)


# speedup vs baseline: 13.2603x; 1.0746x over previous
import functools

import jax
import jax.numpy as jnp
from jax import lax
from jax.experimental import pallas as pl
from jax.experimental.pallas import tpu as pltpu
from jax.experimental.pallas import tpu_sc as plsc

NC = 2
NS = 16
NW = NC * NS
CH = 80
RB = 80


def _sc_aggregate(support, src3, dst3, zblk, nbuf):
    n_nodes, d = support.shape
    cpw = src3.shape[1]
    NBUF = nbuf
    assert cpw > NBUF and n_nodes % RB == 0
    chunks_n = n_nodes // RB
    per_sub = -(-chunks_n // NS)

    mesh = plsc.VectorSubcoreMesh(core_axis_name="c", subcore_axis_name="s")

    @functools.partial(
        pl.kernel,
        out_type=jax.ShapeDtypeStruct((NC, n_nodes, d), jnp.float32),
        mesh=mesh,
        compiler_params=pltpu.CompilerParams(use_tc_tiling_on_sc=False),
        scratch_types=[
            pltpu.VMEM((cpw, CH), jnp.int32),
            pltpu.VMEM((cpw, CH), jnp.int32),
            pltpu.VMEM((NBUF, CH, d), jnp.float32),
            pltpu.VMEM_SHARED((n_nodes, d), jnp.float32),
            [pltpu.SemaphoreType.DMA] * NBUF,
            [pltpu.SemaphoreType.DMA] * NBUF,
            pltpu.SemaphoreType.DMA,
            pltpu.SemaphoreType.DMA,
        ],
    )
    def agg_kernel(support_hbm, src_hbm, dst_hbm, zblk_hbm, out_hbm,
                   srcv, dstv, rows, acc_sh, gsems, ssems, isem, zsem):
        cid = lax.axis_index("c")
        sid = lax.axis_index("s")
        wid = sid * NC + cid

        pltpu.async_copy(src_hbm.at[wid], srcv, isem)
        pltpu.async_copy(dst_hbm.at[wid], dstv, isem)

        @pl.loop(0, per_sub)
        def _(j):
            chunk = sid + j * NS

            @pl.when(chunk < chunks_n)
            def _():
                pltpu.async_copy(zblk_hbm, acc_sh.at[pl.ds(chunk * RB, RB)],
                                 zsem)

        @pl.loop(0, per_sub)
        def _(j):
            chunk = sid + j * NS

            @pl.when(chunk < chunks_n)
            def _():
                pltpu.make_async_copy(
                    zblk_hbm, acc_sh.at[pl.ds(chunk * RB, RB)], zsem).wait()

        pltpu.make_async_copy(src_hbm.at[wid], srcv, isem).wait()
        pltpu.make_async_copy(dst_hbm.at[wid], dstv, isem).wait()

        plsc.subcore_barrier()

        def start_gather(c, b):
            pltpu.async_copy(support_hbm.at[srcv.at[c]], rows.at[b], gsems[b])

        def wait_gather(c, b):
            pltpu.make_async_copy(
                support_hbm.at[srcv.at[c]], rows.at[b], gsems[b]).wait()

        def start_scatter(c, b):
            pltpu.async_copy(rows.at[b], acc_sh.at[dstv.at[c]], ssems[b],
                             add=True)

        def wait_scatter(c, b):
            pltpu.make_async_copy(
                rows.at[b], acc_sh.at[dstv.at[c]], ssems[b]).wait()

        for b in range(NBUF):
            start_gather(b, b)

        n_outer = -(-cpw // NBUF)

        @pl.loop(0, n_outer)
        def _(j):
            for b in range(NBUF):
                c = j * NBUF + b
                prev = (b - 1) % NBUF

                @pl.when(c < cpw)
                def _():
                    wait_gather(c, b)
                    start_scatter(c, b)
                    nxt = c + NBUF - 1

                    @pl.when((c >= 1) & (nxt < cpw))
                    def _():
                        wait_scatter(c - 1, prev)
                        start_gather(nxt, prev)

        for b in range(NBUF):
            c_last = ((cpw - 1 - b) // NBUF) * NBUF + b
            wait_scatter(c_last, b)

        plsc.subcore_barrier()

        @pl.loop(0, per_sub)
        def _(j):
            chunk = sid + j * NS

            @pl.when(chunk < chunks_n)
            def _():
                pltpu.async_copy(acc_sh.at[pl.ds(chunk * RB, RB)],
                                 out_hbm.at[cid, pl.ds(chunk * RB, RB)], zsem)

        @pl.loop(0, per_sub)
        def _(j):
            chunk = sid + j * NS

            @pl.when(chunk < chunks_n)
            def _():
                pltpu.make_async_copy(
                    acc_sh.at[pl.ds(chunk * RB, RB)],
                    out_hbm.at[cid, pl.ds(chunk * RB, RB)], zsem).wait()

    return agg_kernel(support, src3, dst3, zblk)


_BM = 2000


def _tc_matmul(x, w):
    m, k = x.shape
    n = w.shape[1]

    def body(x_ref, w_ref, o_ref):
        o_ref[...] = jnp.dot(x_ref[...], w_ref[...],
                             preferred_element_type=jnp.float32)

    return pl.pallas_call(
        body,
        grid=(m // _BM,),
        in_specs=[pl.BlockSpec((_BM, k), lambda i: (i, 0)),
                  pl.BlockSpec((k, n), lambda i: (0, 0))],
        out_specs=pl.BlockSpec((_BM, n), lambda i: (i, 0)),
        out_shape=jax.ShapeDtypeStruct((m, n), jnp.float32),
    )(x, w)


def _tc_relu_matmul(p, b, w):
    m, k = p.shape[1:]
    n = w.shape[1]

    def body(p0_ref, p1_ref, b_ref, w_ref, o_ref):
        h = jnp.maximum(p0_ref[0] + p1_ref[0] + b_ref[...], 0.0)
        o_ref[...] = jnp.dot(h, w_ref[...], preferred_element_type=jnp.float32)

    return pl.pallas_call(
        body,
        grid=(m // _BM,),
        in_specs=[pl.BlockSpec((1, _BM, k), lambda i: (0, i, 0)),
                  pl.BlockSpec((1, _BM, k), lambda i: (1, i, 0)),
                  pl.BlockSpec((1, k), lambda i: (0, 0)),
                  pl.BlockSpec((k, n), lambda i: (0, 0))],
        out_specs=pl.BlockSpec((_BM, n), lambda i: (i, 0)),
        out_shape=jax.ShapeDtypeStruct((m, n), jnp.float32),
    )(p, p, b.reshape(1, k), w)


def _tc_log_softmax(p, b):
    m, n = p.shape[1:]

    def body(p0_ref, p1_ref, b_ref, o_ref):
        z = p0_ref[0] + p1_ref[0] + b_ref[...]
        zs = z - jnp.max(z, axis=1, keepdims=True)
        o_ref[...] = zs - jnp.log(jnp.sum(jnp.exp(zs), axis=1, keepdims=True))

    return pl.pallas_call(
        body,
        grid=(m // _BM,),
        in_specs=[pl.BlockSpec((1, _BM, n), lambda i: (0, i, 0)),
                  pl.BlockSpec((1, _BM, n), lambda i: (1, i, 0)),
                  pl.BlockSpec((1, n), lambda i: (0, 0))],
        out_specs=pl.BlockSpec((_BM, n), lambda i: (i, 0)),
        out_shape=jax.ShapeDtypeStruct((m, n), jnp.float32),
    )(p, p, b.reshape(1, n))


def kernel(x, edge_index, W1, b1, W2, b2):
    n_edges = edge_index.shape[1]
    cpw = n_edges // (NW * CH)
    assert cpw * NW * CH == n_edges
    src3 = edge_index[0].reshape(NW, cpw, CH)
    dst3 = edge_index[1].reshape(NW, cpw, CH)
    z1 = jnp.zeros((RB, W1.shape[1]), jnp.float32)
    z2 = jnp.zeros((RB, W2.shape[1]), jnp.float32)

    s1 = _tc_matmul(x, W1)
    p1 = _sc_aggregate(s1, src3, dst3, z1, 3)
    s2 = _tc_relu_matmul(p1, b1, W2)
    p2 = _sc_aggregate(s2, src3, dst3, z2, 5)
    return _tc_log_softmax(p2, b2)


# speedup vs baseline: 14.3937x; 1.0855x over previous
import functools

import jax
import jax.numpy as jnp
from jax import lax
from jax.experimental import pallas as pl
from jax.experimental.pallas import tpu as pltpu
from jax.experimental.pallas import tpu_sc as plsc

NC = 2
NS = 16
NW = NC * NS
CH = 80
RB = 80


def _sc_aggregate(support, e4, zblk, nbuf):
    n_nodes, d = support.shape
    cpw = e4.shape[2]
    NBUF = nbuf
    assert cpw > NBUF and n_nodes % RB == 0
    chunks_n = n_nodes // RB
    per_sub = -(-chunks_n // NS)

    mesh = plsc.VectorSubcoreMesh(core_axis_name="c", subcore_axis_name="s")

    @functools.partial(
        pl.kernel,
        out_type=jax.ShapeDtypeStruct((NC, n_nodes, d), jnp.float32),
        mesh=mesh,
        compiler_params=pltpu.CompilerParams(use_tc_tiling_on_sc=False),
        scratch_types=[
            pltpu.VMEM((cpw, CH), jnp.int32),
            pltpu.VMEM((cpw, CH), jnp.int32),
            pltpu.VMEM((NBUF, CH, d), jnp.float32),
            pltpu.VMEM_SHARED((n_nodes, d), jnp.float32),
            [pltpu.SemaphoreType.DMA] * NBUF,
            [pltpu.SemaphoreType.DMA] * NBUF,
            pltpu.SemaphoreType.DMA,
            pltpu.SemaphoreType.DMA,
        ],
    )
    def agg_kernel(support_hbm, e_hbm, zblk_hbm, out_hbm,
                   srcv, dstv, rows, acc_sh, gsems, ssems, isem, zsem):
        cid = lax.axis_index("c")
        sid = lax.axis_index("s")
        wid = sid * NC + cid

        pltpu.async_copy(e_hbm.at[0, wid], srcv, isem)
        pltpu.async_copy(e_hbm.at[1, wid], dstv, isem)

        @pl.loop(0, per_sub)
        def _(j):
            chunk = sid + j * NS

            @pl.when(chunk < chunks_n)
            def _():
                pltpu.async_copy(zblk_hbm, acc_sh.at[pl.ds(chunk * RB, RB)],
                                 zsem)

        @pl.loop(0, per_sub)
        def _(j):
            chunk = sid + j * NS

            @pl.when(chunk < chunks_n)
            def _():
                pltpu.make_async_copy(
                    zblk_hbm, acc_sh.at[pl.ds(chunk * RB, RB)], zsem).wait()

        pltpu.make_async_copy(e_hbm.at[0, wid], srcv, isem).wait()
        pltpu.make_async_copy(e_hbm.at[1, wid], dstv, isem).wait()

        plsc.subcore_barrier()

        def start_gather(c, b):
            pltpu.async_copy(support_hbm.at[srcv.at[c]], rows.at[b], gsems[b])

        def wait_gather(c, b):
            pltpu.make_async_copy(
                support_hbm.at[srcv.at[c]], rows.at[b], gsems[b]).wait()

        def start_scatter(c, b):
            pltpu.async_copy(rows.at[b], acc_sh.at[dstv.at[c]], ssems[b],
                             add=True)

        def wait_scatter(c, b):
            pltpu.make_async_copy(
                rows.at[b], acc_sh.at[dstv.at[c]], ssems[b]).wait()

        for b in range(NBUF):
            start_gather(b, b)

        n_outer = -(-cpw // NBUF)

        @pl.loop(0, n_outer)
        def _(j):
            for b in range(NBUF):
                c = j * NBUF + b
                prev = (b - 1) % NBUF

                @pl.when(c < cpw)
                def _():
                    wait_gather(c, b)
                    start_scatter(c, b)
                    nxt = c + NBUF - 1

                    @pl.when((c >= 1) & (nxt < cpw))
                    def _():
                        wait_scatter(c - 1, prev)
                        start_gather(nxt, prev)

        for b in range(NBUF):
            c_last = ((cpw - 1 - b) // NBUF) * NBUF + b
            wait_scatter(c_last, b)

        plsc.subcore_barrier()

        @pl.loop(0, per_sub)
        def _(j):
            chunk = sid + j * NS

            @pl.when(chunk < chunks_n)
            def _():
                pltpu.async_copy(acc_sh.at[pl.ds(chunk * RB, RB)],
                                 out_hbm.at[cid, pl.ds(chunk * RB, RB)], zsem)

        @pl.loop(0, per_sub)
        def _(j):
            chunk = sid + j * NS

            @pl.when(chunk < chunks_n)
            def _():
                pltpu.make_async_copy(
                    acc_sh.at[pl.ds(chunk * RB, RB)],
                    out_hbm.at[cid, pl.ds(chunk * RB, RB)], zsem).wait()

    return agg_kernel(support, e4, zblk)


_BM = 2000


def _tc_matmul(x, w):
    m, k = x.shape
    n = w.shape[1]

    def body(x_ref, w_ref, o_ref):
        o_ref[...] = jnp.dot(x_ref[...], w_ref[...],
                             preferred_element_type=jnp.float32)

    return pl.pallas_call(
        body,
        grid=(m // _BM,),
        in_specs=[pl.BlockSpec((_BM, k), lambda i: (i, 0)),
                  pl.BlockSpec((k, n), lambda i: (0, 0))],
        out_specs=pl.BlockSpec((_BM, n), lambda i: (i, 0)),
        out_shape=jax.ShapeDtypeStruct((m, n), jnp.float32),
    )(x, w)


def _tc_relu_matmul(p, b, w):
    m, k = p.shape[1:]
    n = w.shape[1]

    def body(p0_ref, p1_ref, b_ref, w_ref, o_ref):
        h = jnp.maximum(p0_ref[0] + p1_ref[0] + b_ref[...], 0.0)
        o_ref[...] = jnp.dot(h, w_ref[...], preferred_element_type=jnp.float32)

    return pl.pallas_call(
        body,
        grid=(m // _BM,),
        in_specs=[pl.BlockSpec((1, _BM, k), lambda i: (0, i, 0)),
                  pl.BlockSpec((1, _BM, k), lambda i: (1, i, 0)),
                  pl.BlockSpec((1, k), lambda i: (0, 0)),
                  pl.BlockSpec((k, n), lambda i: (0, 0))],
        out_specs=pl.BlockSpec((_BM, n), lambda i: (i, 0)),
        out_shape=jax.ShapeDtypeStruct((m, n), jnp.float32),
    )(p, p, b.reshape(1, k), w)


def _tc_log_softmax(p, b):
    m, n = p.shape[1:]

    def body(p0_ref, p1_ref, b_ref, o_ref):
        z = p0_ref[0] + p1_ref[0] + b_ref[...]
        zs = z - jnp.max(z, axis=1, keepdims=True)
        o_ref[...] = zs - jnp.log(jnp.sum(jnp.exp(zs), axis=1, keepdims=True))

    return pl.pallas_call(
        body,
        grid=(m // _BM,),
        in_specs=[pl.BlockSpec((1, _BM, n), lambda i: (0, i, 0)),
                  pl.BlockSpec((1, _BM, n), lambda i: (1, i, 0)),
                  pl.BlockSpec((1, n), lambda i: (0, 0))],
        out_specs=pl.BlockSpec((_BM, n), lambda i: (i, 0)),
        out_shape=jax.ShapeDtypeStruct((m, n), jnp.float32),
    )(p, p, b.reshape(1, n))


def kernel(x, edge_index, W1, b1, W2, b2):
    n_nodes = x.shape[0]
    n_edges = edge_index.shape[1]
    cpw = n_edges // (NW * CH)
    assert cpw * NW * CH == n_edges
    e4 = edge_index.reshape(2, NW, cpw, CH)
    z1 = jnp.zeros((RB, W1.shape[1]), jnp.float32)
    z2 = jnp.zeros((RB, W2.shape[1]), jnp.float32)
    s1 = _tc_matmul(x, W1)
    p1 = _sc_aggregate(s1, e4, z1, 3)
    s2 = _tc_relu_matmul(p1, b1, W2)
    p2 = _sc_aggregate(s2, e4, z2, 5)
    return _tc_log_softmax(p2, b2)


# speedup vs baseline: 15.0538x; 1.0459x over previous
import functools

import jax
import jax.numpy as jnp
from jax import lax
from jax.experimental import pallas as pl
from jax.experimental.pallas import tpu as pltpu
from jax.experimental.pallas import tpu_sc as plsc

NC = 2
NS = 16
NW = NC * NS
CH = 80
RB = 80


def _sc_aggregate(support, e4, zblk, nbuf, out_d=None):
    n_nodes, d = support.shape
    out_d = out_d or d
    cpw = e4.shape[2]
    NBUF = nbuf
    assert cpw > NBUF and n_nodes % RB == 0
    chunks_n = n_nodes // RB
    per_sub = -(-chunks_n // NS)

    mesh = plsc.VectorSubcoreMesh(core_axis_name="c", subcore_axis_name="s")

    @functools.partial(
        pl.kernel,
        out_type=jax.ShapeDtypeStruct((NC, n_nodes, out_d), jnp.float32),
        mesh=mesh,
        compiler_params=pltpu.CompilerParams(use_tc_tiling_on_sc=False),
        scratch_types=[
            pltpu.VMEM((cpw, CH), jnp.int32),
            pltpu.VMEM((cpw, CH), jnp.int32),
            pltpu.VMEM((NBUF, CH, d), jnp.float32),
            pltpu.VMEM_SHARED((n_nodes, d), jnp.float32),
            [pltpu.SemaphoreType.DMA] * NBUF,
            [pltpu.SemaphoreType.DMA] * NBUF,
            pltpu.SemaphoreType.DMA,
            pltpu.SemaphoreType.DMA,
        ],
    )
    def agg_kernel(support_hbm, e_hbm, zblk_hbm, out_hbm,
                   srcv, dstv, rows, acc_sh, gsems, ssems, isem, zsem):
        cid = lax.axis_index("c")
        sid = lax.axis_index("s")
        wid = sid * NC + cid

        pltpu.async_copy(e_hbm.at[0, wid], srcv, isem)
        pltpu.async_copy(e_hbm.at[1, wid], dstv, isem)

        @pl.loop(0, per_sub)
        def _(j):
            chunk = sid + j * NS

            @pl.when(chunk < chunks_n)
            def _():
                pltpu.async_copy(zblk_hbm, acc_sh.at[pl.ds(chunk * RB, RB)],
                                 zsem)

        @pl.loop(0, per_sub)
        def _(j):
            chunk = sid + j * NS

            @pl.when(chunk < chunks_n)
            def _():
                pltpu.make_async_copy(
                    zblk_hbm, acc_sh.at[pl.ds(chunk * RB, RB)], zsem).wait()

        pltpu.make_async_copy(e_hbm.at[0, wid], srcv, isem).wait()
        pltpu.make_async_copy(e_hbm.at[1, wid], dstv, isem).wait()

        plsc.subcore_barrier()

        def start_gather(c, b):
            pltpu.async_copy(support_hbm.at[srcv.at[c]], rows.at[b], gsems[b])

        def wait_gather(c, b):
            pltpu.make_async_copy(
                support_hbm.at[srcv.at[c]], rows.at[b], gsems[b]).wait()

        def start_scatter(c, b):
            pltpu.async_copy(rows.at[b], acc_sh.at[dstv.at[c]], ssems[b],
                             add=True)

        def wait_scatter(c, b):
            pltpu.make_async_copy(
                rows.at[b], acc_sh.at[dstv.at[c]], ssems[b]).wait()

        for b in range(NBUF):
            start_gather(b, b)

        n_outer = -(-cpw // NBUF)

        @pl.loop(0, n_outer)
        def _(j):
            for b in range(NBUF):
                c = j * NBUF + b
                prev = (b - 1) % NBUF

                @pl.when(c < cpw)
                def _():
                    wait_gather(c, b)
                    start_scatter(c, b)
                    nxt = c + NBUF - 1

                    @pl.when((c >= 1) & (nxt < cpw))
                    def _():
                        wait_scatter(c - 1, prev)
                        start_gather(nxt, prev)

        for b in range(NBUF):
            c_last = ((cpw - 1 - b) // NBUF) * NBUF + b
            wait_scatter(c_last, b)

        plsc.subcore_barrier()

        def out_dst(chunk):
            if out_d == d:
                return out_hbm.at[cid, pl.ds(chunk * RB, RB)]
            return out_hbm.at[cid, pl.ds(chunk * RB, RB), pl.ds(0, d)]

        @pl.loop(0, per_sub)
        def _(j):
            chunk = sid + j * NS

            @pl.when(chunk < chunks_n)
            def _():
                pltpu.async_copy(acc_sh.at[pl.ds(chunk * RB, RB)],
                                 out_dst(chunk), zsem)

        @pl.loop(0, per_sub)
        def _(j):
            chunk = sid + j * NS

            @pl.when(chunk < chunks_n)
            def _():
                pltpu.make_async_copy(
                    acc_sh.at[pl.ds(chunk * RB, RB)],
                    out_dst(chunk), zsem).wait()

    return agg_kernel(support, e4, zblk)


_BM = 2000


def _tc_matmul(x, w):
    m, k = x.shape
    n = w.shape[1]

    def body(x_ref, w_ref, o_ref):
        o_ref[...] = jnp.dot(x_ref[...], w_ref[...],
                             preferred_element_type=jnp.float32)

    return pl.pallas_call(
        body,
        grid=(m // _BM,),
        in_specs=[pl.BlockSpec((_BM, k), lambda i: (i, 0)),
                  pl.BlockSpec((k, n), lambda i: (0, 0))],
        out_specs=pl.BlockSpec((_BM, n), lambda i: (i, 0)),
        out_shape=jax.ShapeDtypeStruct((m, n), jnp.float32),
    )(x, w)


def _tc_relu_matmul(p, b, w):
    m, k = p.shape[1:]
    n = w.shape[1]

    def body(p0_ref, p1_ref, b_ref, w_ref, o_ref):
        h = jnp.maximum(p0_ref[0] + p1_ref[0] + b_ref[...], 0.0)
        o_ref[...] = jnp.dot(h, w_ref[...], preferred_element_type=jnp.float32)

    return pl.pallas_call(
        body,
        grid=(m // _BM,),
        in_specs=[pl.BlockSpec((1, _BM, k), lambda i: (0, i, 0)),
                  pl.BlockSpec((1, _BM, k), lambda i: (1, i, 0)),
                  pl.BlockSpec((1, k), lambda i: (0, 0)),
                  pl.BlockSpec((k, n), lambda i: (0, 0))],
        out_specs=pl.BlockSpec((_BM, n), lambda i: (i, 0)),
        out_shape=jax.ShapeDtypeStruct((m, n), jnp.float32),
    )(p, p, b.reshape(1, k), w)


def _tc_log_softmax(p, b):
    m, dpad = p.shape[1:]
    n = b.shape[0]

    def body(p0_ref, p1_ref, b_ref, o_ref):
        z = p0_ref[0, :, :n] + p1_ref[0, :, :n] + b_ref[...]
        zs = z - jnp.max(z, axis=1, keepdims=True)
        o_ref[...] = zs - jnp.log(jnp.sum(jnp.exp(zs), axis=1, keepdims=True))

    return pl.pallas_call(
        body,
        grid=(m // _BM,),
        in_specs=[pl.BlockSpec((1, _BM, dpad), lambda i: (0, i, 0)),
                  pl.BlockSpec((1, _BM, dpad), lambda i: (1, i, 0)),
                  pl.BlockSpec((1, n), lambda i: (0, 0))],
        out_specs=pl.BlockSpec((_BM, n), lambda i: (i, 0)),
        out_shape=jax.ShapeDtypeStruct((m, n), jnp.float32),
    )(p, p, b.reshape(1, n))


def kernel(x, edge_index, W1, b1, W2, b2):
    n_nodes = x.shape[0]
    n_edges = edge_index.shape[1]
    cpw = n_edges // (NW * CH)
    assert cpw * NW * CH == n_edges
    e4 = edge_index.reshape(2, NW, cpw, CH)
    z1 = jnp.zeros((RB, W1.shape[1]), jnp.float32)
    z2 = jnp.zeros((RB, W2.shape[1]), jnp.float32)
    s1 = _tc_matmul(x, W1)
    p1 = _sc_aggregate(s1, e4, z1, 3)
    s2 = _tc_relu_matmul(p1, b1, W2)
    p2 = _sc_aggregate(s2, e4, z2, 8, out_d=128)
    return _tc_log_softmax(p2, b2)


# speedup vs baseline: 15.1102x; 1.0037x over previous
import functools

import jax
import jax.numpy as jnp
from jax import lax
from jax.experimental import pallas as pl
from jax.experimental.pallas import tpu as pltpu
from jax.experimental.pallas import tpu_sc as plsc

NC = 2
NS = 16
NW = NC * NS
CH = 80
RB = 80


def _sc_aggregate(support, e4, zblk, nbuf, out_d=None, n_rows=None):
    sup_rows, d = support.shape
    n_nodes = n_rows or sup_rows
    out_d = out_d or d
    cpw = e4.shape[2]
    NBUF = nbuf
    assert cpw > NBUF and n_nodes % RB == 0
    chunks_n = n_nodes // RB
    per_sub = -(-chunks_n // NS)

    mesh = plsc.VectorSubcoreMesh(core_axis_name="c", subcore_axis_name="s")

    @functools.partial(
        pl.kernel,
        out_type=jax.ShapeDtypeStruct((NC, n_nodes, out_d), jnp.float32),
        mesh=mesh,
        compiler_params=pltpu.CompilerParams(use_tc_tiling_on_sc=False),
        scratch_types=[
            pltpu.VMEM((cpw, CH), jnp.int32),
            pltpu.VMEM((cpw, CH), jnp.int32),
            pltpu.VMEM((NBUF, CH, d), jnp.float32),
            pltpu.VMEM_SHARED((n_nodes, d), jnp.float32),
            [pltpu.SemaphoreType.DMA] * NBUF,
            [pltpu.SemaphoreType.DMA] * NBUF,
            pltpu.SemaphoreType.DMA,
            pltpu.SemaphoreType.DMA,
        ],
    )
    def agg_kernel(support_hbm, e_hbm, zblk_hbm, out_hbm,
                   srcv, dstv, rows, acc_sh, gsems, ssems, isem, zsem):
        cid = lax.axis_index("c")
        sid = lax.axis_index("s")
        wid = sid * NC + cid

        pltpu.async_copy(e_hbm.at[0, wid], srcv, isem)
        pltpu.async_copy(e_hbm.at[1, wid], dstv, isem)

        @pl.loop(0, per_sub)
        def _(j):
            chunk = sid + j * NS

            @pl.when(chunk < chunks_n)
            def _():
                pltpu.async_copy(zblk_hbm, acc_sh.at[pl.ds(chunk * RB, RB)],
                                 zsem)

        @pl.loop(0, per_sub)
        def _(j):
            chunk = sid + j * NS

            @pl.when(chunk < chunks_n)
            def _():
                pltpu.make_async_copy(
                    zblk_hbm, acc_sh.at[pl.ds(chunk * RB, RB)], zsem).wait()

        pltpu.make_async_copy(e_hbm.at[0, wid], srcv, isem).wait()
        pltpu.make_async_copy(e_hbm.at[1, wid], dstv, isem).wait()

        plsc.subcore_barrier()

        def start_gather(c, b):
            pltpu.async_copy(support_hbm.at[srcv.at[c]], rows.at[b], gsems[b])

        def wait_gather(c, b):
            pltpu.make_async_copy(
                support_hbm.at[srcv.at[c]], rows.at[b], gsems[b]).wait()

        def start_scatter(c, b):
            pltpu.async_copy(rows.at[b], acc_sh.at[dstv.at[c]], ssems[b],
                             add=True)

        def wait_scatter(c, b):
            pltpu.make_async_copy(
                rows.at[b], acc_sh.at[dstv.at[c]], ssems[b]).wait()

        for b in range(NBUF):
            start_gather(b, b)

        n_outer = -(-cpw // NBUF)

        @pl.loop(0, n_outer)
        def _(j):
            for b in range(NBUF):
                c = j * NBUF + b
                prev = (b - 1) % NBUF

                @pl.when(c < cpw)
                def _():
                    wait_gather(c, b)
                    start_scatter(c, b)
                    nxt = c + NBUF - 1

                    @pl.when((c >= 1) & (nxt < cpw))
                    def _():
                        wait_scatter(c - 1, prev)
                        start_gather(nxt, prev)

        for b in range(NBUF):
            c_last = ((cpw - 1 - b) // NBUF) * NBUF + b
            wait_scatter(c_last, b)

        plsc.subcore_barrier()

        def out_dst(chunk):
            if out_d == d:
                return out_hbm.at[cid, pl.ds(chunk * RB, RB)]
            return out_hbm.at[cid, pl.ds(chunk * RB, RB), pl.ds(0, d)]

        @pl.loop(0, per_sub)
        def _(j):
            chunk = sid + j * NS

            @pl.when(chunk < chunks_n)
            def _():
                pltpu.async_copy(acc_sh.at[pl.ds(chunk * RB, RB)],
                                 out_dst(chunk), zsem)

        @pl.loop(0, per_sub)
        def _(j):
            chunk = sid + j * NS

            @pl.when(chunk < chunks_n)
            def _():
                pltpu.make_async_copy(
                    acc_sh.at[pl.ds(chunk * RB, RB)],
                    out_dst(chunk), zsem).wait()

    return agg_kernel(support, e4, zblk)


_BM = 2000


def _tc_matmul(x, w):
    m, k = x.shape
    n = w.shape[1]

    def body(x_ref, w_ref, o_ref):
        o_ref[...] = jnp.dot(x_ref[...], w_ref[...],
                             preferred_element_type=jnp.float32)

    return pl.pallas_call(
        body,
        grid=(m // _BM,),
        in_specs=[pl.BlockSpec((_BM, k), lambda i: (i, 0)),
                  pl.BlockSpec((k, n), lambda i: (0, 0))],
        out_specs=pl.BlockSpec((_BM, n), lambda i: (i, 0)),
        out_shape=jax.ShapeDtypeStruct((m, n), jnp.float32),
    )(x, w)


def _tc_relu_matmul(p, b, w):
    m, k = p.shape[1:]
    n = w.shape[1]

    def body(p0_ref, p1_ref, b_ref, w_ref, o_ref):
        h = jnp.maximum(p0_ref[0] + p1_ref[0] + b_ref[...], 0.0)
        o_ref[:, :n] = jnp.dot(h, w_ref[...],
                               preferred_element_type=jnp.float32)

    return pl.pallas_call(
        body,
        grid=(m // _BM,),
        in_specs=[pl.BlockSpec((1, _BM, k), lambda i: (0, i, 0)),
                  pl.BlockSpec((1, _BM, k), lambda i: (1, i, 0)),
                  pl.BlockSpec((1, k), lambda i: (0, 0)),
                  pl.BlockSpec((k, n), lambda i: (0, 0))],
        out_specs=pl.BlockSpec((_BM, 2 * n), lambda i: (i, 0)),
        out_shape=jax.ShapeDtypeStruct((m, 2 * n), jnp.float32),
    )(p, p, b.reshape(1, k), w)


def _tc_log_softmax(p, b):
    m, dpad = p.shape[1:]
    n = b.shape[0]

    def body(p0_ref, p1_ref, b_ref, o_ref):
        z = p0_ref[0, :, :n] + p1_ref[0, :, :n] + b_ref[...]
        zs = z - jnp.max(z, axis=1, keepdims=True)
        o_ref[...] = zs - jnp.log(jnp.sum(jnp.exp(zs), axis=1, keepdims=True))

    return pl.pallas_call(
        body,
        grid=(m // _BM,),
        in_specs=[pl.BlockSpec((1, _BM, dpad), lambda i: (0, i, 0)),
                  pl.BlockSpec((1, _BM, dpad), lambda i: (1, i, 0)),
                  pl.BlockSpec((1, n), lambda i: (0, 0))],
        out_specs=pl.BlockSpec((_BM, n), lambda i: (i, 0)),
        out_shape=jax.ShapeDtypeStruct((m, n), jnp.float32),
    )(p, p, b.reshape(1, n))


def kernel(x, edge_index, W1, b1, W2, b2):
    n_nodes = x.shape[0]
    n_edges = edge_index.shape[1]
    cpw = n_edges // (NW * CH)
    assert cpw * NW * CH == n_edges
    e4 = edge_index.reshape(2, NW, cpw, CH)
    e4b = jnp.stack([edge_index[0] * 2, edge_index[1]]).reshape(
        2, NW, cpw, CH)
    z1 = jnp.zeros((RB, W1.shape[1]), jnp.float32)
    z2 = jnp.zeros((RB, W2.shape[1]), jnp.float32)
    s1 = _tc_matmul(x, W1)
    p1 = _sc_aggregate(s1, e4, z1, 3)
    s2p = _tc_relu_matmul(p1, b1, W2)
    s2 = s2p.reshape(2 * n_nodes, W2.shape[1])
    p2 = _sc_aggregate(s2, e4b, z2, 8, out_d=128,
                       n_rows=n_nodes)
    return _tc_log_softmax(p2, b2)


# speedup vs baseline: 15.3913x; 1.0186x over previous
import functools

import jax
import jax.numpy as jnp
from jax import lax
from jax.experimental import pallas as pl
from jax.experimental.pallas import tpu as pltpu
from jax.experimental.pallas import tpu_sc as plsc

NC = 2
NS = 16
NW = NC * NS
CH = 80
RB = 80


def _sc_aggregate(support, e4, zblk, nbuf, out_d=None, n_rows=None,
                  src_scale=1):
    sup_rows, d = support.shape
    n_nodes = n_rows or sup_rows
    out_d = out_d or d
    cpw = e4.shape[2]
    NBUF = nbuf
    assert cpw > NBUF and n_nodes % RB == 0
    chunks_n = n_nodes // RB
    per_sub = -(-chunks_n // NS)

    mesh = plsc.VectorSubcoreMesh(core_axis_name="c", subcore_axis_name="s")

    @functools.partial(
        pl.kernel,
        out_type=jax.ShapeDtypeStruct((NC, n_nodes, out_d), jnp.float32),
        mesh=mesh,
        compiler_params=pltpu.CompilerParams(use_tc_tiling_on_sc=False),
        scratch_types=[
            pltpu.VMEM((cpw, CH), jnp.int32),
            pltpu.VMEM((cpw, CH), jnp.int32),
            pltpu.VMEM((NBUF, CH, d), jnp.float32),
            pltpu.VMEM_SHARED((n_nodes, d), jnp.float32),
            [pltpu.SemaphoreType.DMA] * NBUF,
            [pltpu.SemaphoreType.DMA] * NBUF,
            pltpu.SemaphoreType.DMA,
            pltpu.SemaphoreType.DMA,
        ],
    )
    def agg_kernel(support_hbm, e_hbm, zblk_hbm, out_hbm,
                   srcv, dstv, rows, acc_sh, gsems, ssems, isem, zsem):
        cid = lax.axis_index("c")
        sid = lax.axis_index("s")
        wid = sid * NC + cid

        pltpu.async_copy(e_hbm.at[0, wid], srcv, isem)
        pltpu.async_copy(e_hbm.at[1, wid], dstv, isem)

        @pl.loop(0, per_sub)
        def _(j):
            chunk = sid + j * NS

            @pl.when(chunk < chunks_n)
            def _():
                pltpu.async_copy(zblk_hbm, acc_sh.at[pl.ds(chunk * RB, RB)],
                                 zsem)

        @pl.loop(0, per_sub)
        def _(j):
            chunk = sid + j * NS

            @pl.when(chunk < chunks_n)
            def _():
                pltpu.make_async_copy(
                    zblk_hbm, acc_sh.at[pl.ds(chunk * RB, RB)], zsem).wait()

        pltpu.make_async_copy(e_hbm.at[0, wid], srcv, isem).wait()
        pltpu.make_async_copy(e_hbm.at[1, wid], dstv, isem).wait()

        plsc.subcore_barrier()

        def scale_src(c):
            if src_scale != 1:
                for i in range(CH // 16):
                    srcv[c, pl.ds(i * 16, 16)] = (
                        srcv[c, pl.ds(i * 16, 16)] * src_scale)

        def start_gather(c, b):
            pltpu.async_copy(support_hbm.at[srcv.at[c]], rows.at[b], gsems[b])

        def wait_gather(c, b):
            pltpu.make_async_copy(
                support_hbm.at[srcv.at[c]], rows.at[b], gsems[b]).wait()

        def start_scatter(c, b):
            pltpu.async_copy(rows.at[b], acc_sh.at[dstv.at[c]], ssems[b],
                             add=True)

        def wait_scatter(c, b):
            pltpu.make_async_copy(
                rows.at[b], acc_sh.at[dstv.at[c]], ssems[b]).wait()

        for b in range(NBUF):
            scale_src(b)
            start_gather(b, b)

        n_outer = -(-cpw // NBUF)

        @pl.loop(0, n_outer)
        def _(j):
            for b in range(NBUF):
                c = j * NBUF + b
                prev = (b - 1) % NBUF

                @pl.when(c < cpw)
                def _():
                    wait_gather(c, b)
                    start_scatter(c, b)
                    nxt = c + NBUF - 1

                    @pl.when((c >= 1) & (nxt < cpw))
                    def _():
                        wait_scatter(c - 1, prev)
                        scale_src(nxt)
                        start_gather(nxt, prev)

        for b in range(NBUF):
            c_last = ((cpw - 1 - b) // NBUF) * NBUF + b
            wait_scatter(c_last, b)

        plsc.subcore_barrier()

        def out_dst(chunk):
            if out_d == d:
                return out_hbm.at[cid, pl.ds(chunk * RB, RB)]
            return out_hbm.at[cid, pl.ds(chunk * RB, RB), pl.ds(0, d)]

        @pl.loop(0, per_sub)
        def _(j):
            chunk = sid + j * NS

            @pl.when(chunk < chunks_n)
            def _():
                pltpu.async_copy(acc_sh.at[pl.ds(chunk * RB, RB)],
                                 out_dst(chunk), zsem)

        @pl.loop(0, per_sub)
        def _(j):
            chunk = sid + j * NS

            @pl.when(chunk < chunks_n)
            def _():
                pltpu.make_async_copy(
                    acc_sh.at[pl.ds(chunk * RB, RB)],
                    out_dst(chunk), zsem).wait()

    return agg_kernel(support, e4, zblk)


_BM = 2000


def _tc_matmul(x, w):
    m, k = x.shape
    n = w.shape[1]

    def body(x_ref, w_ref, o_ref):
        o_ref[...] = jnp.dot(x_ref[...], w_ref[...],
                             preferred_element_type=jnp.float32)

    return pl.pallas_call(
        body,
        grid=(m // _BM,),
        in_specs=[pl.BlockSpec((_BM, k), lambda i: (i, 0)),
                  pl.BlockSpec((k, n), lambda i: (0, 0))],
        out_specs=pl.BlockSpec((_BM, n), lambda i: (i, 0)),
        out_shape=jax.ShapeDtypeStruct((m, n), jnp.float32),
    )(x, w)


def _tc_relu_matmul(p, b, w):
    m, k = p.shape[1:]
    n = w.shape[1]

    def body(p0_ref, p1_ref, b_ref, w_ref, o_ref):
        h = jnp.maximum(p0_ref[0] + p1_ref[0] + b_ref[...], 0.0)
        o_ref[:, :n] = jnp.dot(h, w_ref[...],
                               preferred_element_type=jnp.float32)

    return pl.pallas_call(
        body,
        grid=(m // _BM,),
        in_specs=[pl.BlockSpec((1, _BM, k), lambda i: (0, i, 0)),
                  pl.BlockSpec((1, _BM, k), lambda i: (1, i, 0)),
                  pl.BlockSpec((1, k), lambda i: (0, 0)),
                  pl.BlockSpec((k, n), lambda i: (0, 0))],
        out_specs=pl.BlockSpec((_BM, 2 * n), lambda i: (i, 0)),
        out_shape=jax.ShapeDtypeStruct((m, 2 * n), jnp.float32),
    )(p, p, b.reshape(1, k), w)


def _tc_log_softmax(p, b):
    m, dpad = p.shape[1:]
    n = b.shape[0]

    def body(p0_ref, p1_ref, b_ref, o_ref):
        z = p0_ref[0, :, :n] + p1_ref[0, :, :n] + b_ref[...]
        zs = z - jnp.max(z, axis=1, keepdims=True)
        o_ref[...] = zs - jnp.log(jnp.sum(jnp.exp(zs), axis=1, keepdims=True))

    return pl.pallas_call(
        body,
        grid=(m // _BM,),
        in_specs=[pl.BlockSpec((1, _BM, dpad), lambda i: (0, i, 0)),
                  pl.BlockSpec((1, _BM, dpad), lambda i: (1, i, 0)),
                  pl.BlockSpec((1, n), lambda i: (0, 0))],
        out_specs=pl.BlockSpec((_BM, n), lambda i: (i, 0)),
        out_shape=jax.ShapeDtypeStruct((m, n), jnp.float32),
    )(p, p, b.reshape(1, n))


def kernel(x, edge_index, W1, b1, W2, b2):
    n_nodes = x.shape[0]
    n_edges = edge_index.shape[1]
    cpw = n_edges // (NW * CH)
    assert cpw * NW * CH == n_edges
    e4 = edge_index.reshape(2, NW, cpw, CH)
    z1 = jnp.zeros((RB, W1.shape[1]), jnp.float32)
    z2 = jnp.zeros((RB, W2.shape[1]), jnp.float32)
    s1 = _tc_matmul(x, W1)
    p1 = _sc_aggregate(s1, e4, z1, 3)
    s2p = _tc_relu_matmul(p1, b1, W2)
    s2 = s2p.reshape(2 * n_nodes, W2.shape[1])
    p2 = _sc_aggregate(s2, e4, z2, 8, out_d=128,
                       n_rows=n_nodes, src_scale=2)
    return _tc_log_softmax(p2, b2)


# speedup vs baseline: 15.4498x; 1.0038x over previous
import functools

import jax
import jax.numpy as jnp
from jax import lax
from jax.experimental import pallas as pl
from jax.experimental.pallas import tpu as pltpu
from jax.experimental.pallas import tpu_sc as plsc

NC = 2
NS = 16
NW = NC * NS
CH = 80
RB = 80


def _sc_aggregate(support, e4, zblk, nbuf, out_d=None, n_rows=None,
                  src_scale=1):
    sup_rows, d = support.shape
    n_nodes = n_rows or sup_rows
    out_d = out_d or d
    cpw = e4.shape[2]
    NBUF = nbuf
    assert cpw > NBUF and n_nodes % RB == 0
    chunks_n = n_nodes // RB
    per_sub = -(-chunks_n // NS)

    mesh = plsc.VectorSubcoreMesh(core_axis_name="c", subcore_axis_name="s")

    @functools.partial(
        pl.kernel,
        out_type=jax.ShapeDtypeStruct((NC, n_nodes, out_d), jnp.float32),
        mesh=mesh,
        compiler_params=pltpu.CompilerParams(use_tc_tiling_on_sc=False),
        scratch_types=[
            pltpu.VMEM((cpw, CH), jnp.int32),
            pltpu.VMEM((cpw, CH), jnp.int32),
            pltpu.VMEM((NBUF, CH, d), jnp.float32),
            pltpu.VMEM_SHARED((n_nodes, d), jnp.float32),
            [pltpu.SemaphoreType.DMA] * NBUF,
            [pltpu.SemaphoreType.DMA] * NBUF,
            pltpu.SemaphoreType.DMA,
            pltpu.SemaphoreType.DMA,
        ],
    )
    def agg_kernel(support_hbm, e_hbm, zblk_hbm, out_hbm,
                   srcv, dstv, rows, acc_sh, gsems, ssems, isem, zsem):
        cid = lax.axis_index("c")
        sid = lax.axis_index("s")
        wid = sid * NC + cid

        pltpu.async_copy(e_hbm.at[0, wid], srcv, isem)
        pltpu.async_copy(e_hbm.at[1, wid], dstv, isem)

        @pl.loop(0, per_sub)
        def _(j):
            chunk = sid + j * NS

            @pl.when(chunk < chunks_n)
            def _():
                pltpu.async_copy(zblk_hbm, acc_sh.at[pl.ds(chunk * RB, RB)],
                                 zsem)

        pltpu.make_async_copy(e_hbm.at[0, wid], srcv, isem).wait()
        pltpu.make_async_copy(e_hbm.at[1, wid], dstv, isem).wait()

        def scale_src(c):
            if src_scale != 1:
                for i in range(CH // 16):
                    srcv[c, pl.ds(i * 16, 16)] = (
                        srcv[c, pl.ds(i * 16, 16)] * src_scale)

        def start_gather(c, b):
            pltpu.async_copy(support_hbm.at[srcv.at[c]], rows.at[b], gsems[b])

        def wait_gather(c, b):
            pltpu.make_async_copy(
                support_hbm.at[srcv.at[c]], rows.at[b], gsems[b]).wait()

        def start_scatter(c, b):
            pltpu.async_copy(rows.at[b], acc_sh.at[dstv.at[c]], ssems[b],
                             add=True)

        def wait_scatter(c, b):
            pltpu.make_async_copy(
                rows.at[b], acc_sh.at[dstv.at[c]], ssems[b]).wait()

        for b in range(NBUF):
            scale_src(b)
            start_gather(b, b)

        @pl.loop(0, per_sub)
        def _(j):
            chunk = sid + j * NS

            @pl.when(chunk < chunks_n)
            def _():
                pltpu.make_async_copy(
                    zblk_hbm, acc_sh.at[pl.ds(chunk * RB, RB)], zsem).wait()

        plsc.subcore_barrier()

        n_outer = -(-cpw // NBUF)

        @pl.loop(0, n_outer)
        def _(j):
            for b in range(NBUF):
                c = j * NBUF + b
                prev = (b - 1) % NBUF

                @pl.when(c < cpw)
                def _():
                    wait_gather(c, b)
                    start_scatter(c, b)
                    nxt = c + NBUF - 1

                    @pl.when((c >= 1) & (nxt < cpw))
                    def _():
                        wait_scatter(c - 1, prev)
                        scale_src(nxt)
                        start_gather(nxt, prev)

        for b in range(NBUF):
            c_last = ((cpw - 1 - b) // NBUF) * NBUF + b
            wait_scatter(c_last, b)

        plsc.subcore_barrier()

        def out_dst(chunk):
            if out_d == d:
                return out_hbm.at[cid, pl.ds(chunk * RB, RB)]
            return out_hbm.at[cid, pl.ds(chunk * RB, RB), pl.ds(0, d)]

        @pl.loop(0, per_sub)
        def _(j):
            chunk = sid + j * NS

            @pl.when(chunk < chunks_n)
            def _():
                pltpu.async_copy(acc_sh.at[pl.ds(chunk * RB, RB)],
                                 out_dst(chunk), zsem)

        @pl.loop(0, per_sub)
        def _(j):
            chunk = sid + j * NS

            @pl.when(chunk < chunks_n)
            def _():
                pltpu.make_async_copy(
                    acc_sh.at[pl.ds(chunk * RB, RB)],
                    out_dst(chunk), zsem).wait()

    return agg_kernel(support, e4, zblk)


_BM = 2000


def _tc_matmul(x, w):
    m, k = x.shape
    n = w.shape[1]

    def body(x_ref, w_ref, o_ref):
        o_ref[...] = jnp.dot(x_ref[...], w_ref[...],
                             preferred_element_type=jnp.float32)

    return pl.pallas_call(
        body,
        grid=(m // _BM,),
        in_specs=[pl.BlockSpec((_BM, k), lambda i: (i, 0)),
                  pl.BlockSpec((k, n), lambda i: (0, 0))],
        out_specs=pl.BlockSpec((_BM, n), lambda i: (i, 0)),
        out_shape=jax.ShapeDtypeStruct((m, n), jnp.float32),
    )(x, w)


def _tc_relu_matmul(p, b, w):
    m, k = p.shape[1:]
    n = w.shape[1]

    def body(p0_ref, p1_ref, b_ref, w_ref, o_ref):
        h = jnp.maximum(p0_ref[0] + p1_ref[0] + b_ref[...], 0.0)
        o_ref[:, :n] = jnp.dot(h, w_ref[...],
                               preferred_element_type=jnp.float32)

    return pl.pallas_call(
        body,
        grid=(m // _BM,),
        in_specs=[pl.BlockSpec((1, _BM, k), lambda i: (0, i, 0)),
                  pl.BlockSpec((1, _BM, k), lambda i: (1, i, 0)),
                  pl.BlockSpec((1, k), lambda i: (0, 0)),
                  pl.BlockSpec((k, n), lambda i: (0, 0))],
        out_specs=pl.BlockSpec((_BM, 2 * n), lambda i: (i, 0)),
        out_shape=jax.ShapeDtypeStruct((m, 2 * n), jnp.float32),
    )(p, p, b.reshape(1, k), w)


def _tc_log_softmax(p, b):
    m, dpad = p.shape[1:]
    n = b.shape[0]

    def body(p0_ref, p1_ref, b_ref, o_ref):
        z = p0_ref[0, :, :n] + p1_ref[0, :, :n] + b_ref[...]
        zs = z - jnp.max(z, axis=1, keepdims=True)
        o_ref[...] = zs - jnp.log(jnp.sum(jnp.exp(zs), axis=1, keepdims=True))

    return pl.pallas_call(
        body,
        grid=(m // _BM,),
        in_specs=[pl.BlockSpec((1, _BM, dpad), lambda i: (0, i, 0)),
                  pl.BlockSpec((1, _BM, dpad), lambda i: (1, i, 0)),
                  pl.BlockSpec((1, n), lambda i: (0, 0))],
        out_specs=pl.BlockSpec((_BM, n), lambda i: (i, 0)),
        out_shape=jax.ShapeDtypeStruct((m, n), jnp.float32),
    )(p, p, b.reshape(1, n))


def kernel(x, edge_index, W1, b1, W2, b2):
    n_nodes = x.shape[0]
    n_edges = edge_index.shape[1]
    cpw = n_edges // (NW * CH)
    assert cpw * NW * CH == n_edges
    e4 = edge_index.reshape(2, NW, cpw, CH)
    z1 = jnp.zeros((RB, W1.shape[1]), jnp.float32)
    z2 = jnp.zeros((RB, W2.shape[1]), jnp.float32)
    s1 = _tc_matmul(x, W1)
    p1 = _sc_aggregate(s1, e4, z1, 3)
    s2p = _tc_relu_matmul(p1, b1, W2)
    s2 = s2p.reshape(2 * n_nodes, W2.shape[1])
    p2 = _sc_aggregate(s2, e4, z2, 8, out_d=128,
                       n_rows=n_nodes, src_scale=2)
    return _tc_log_softmax(p2, b2)
